# baseline (device time: 220643 ns/iter reference)
import jax
import jax.numpy as jnp
from jax import lax
from jax.experimental import pallas as pl
from jax.experimental.pallas import tpu as pltpu

N_DEV = 16
M = 4096
N_OUT = 2048
CHUNK = M // N_DEV
HALF = CHUNK // 2
N_CHAINS = 4
NCOL = N_OUT // N_CHAINS

RS_STEPS = N_DEV - 1
NSLOT = 4
T_TOTAL = 2 * (N_DEV - 1)

_GELU_C = 0.7978845608028654


def _gelu(y):
    return 0.5 * y * (1.0 + jnp.tanh(_GELU_C * (y + 0.044715 * y * y * y)))


def kernel(x, w_mat):
    def body(x_ref, w_ref, out_ref, *scratch):
        bufs = scratch[:4 * N_CHAINS]
        dma_sems = scratch[4 * N_CHAINS:8 * N_CHAINS]
        credits = scratch[8 * N_CHAINS:]

        my = lax.axis_index("i")
        left = lax.rem(my + N_DEV - 1, N_DEV)
        right = lax.rem(my + 1, N_DEV)

        barrier_sem = pltpu.get_barrier_semaphore()
        for nbr in (left, right):
            pl.semaphore_signal(
                barrier_sem, inc=1,
                device_id=(nbr,), device_id_type=pl.DeviceIdType.MESH,
            )
        pl.semaphore_wait(barrier_sem, 2)

        out_ref[...] = jnp.dot(
            x_ref[...].astype(jnp.bfloat16),
            w_ref[...].astype(jnp.bfloat16),
            preferred_element_type=jnp.float32,
        )

        def top(c):
            return pl.ds(c * CHUNK, HALF)

        def bot(c):
            return pl.ds(c * CHUNK + HALF, HALF)

        def mod(v):
            return lax.rem(v + 2 * N_DEV, N_DEV)

        def c_send_r(t):
            return mod(my - t) if t < RS_STEPS else mod(my + 1 - (t - RS_STEPS))

        def c_recv_r(t):
            return mod(my - t - 1) if t < RS_STEPS else mod(my - (t - RS_STEPS))

        def c_send_l(t):
            return mod(my + t) if t < RS_STEPS else mod(my - 1 + (t - RS_STEPS))

        def c_recv_l(t):
            return mod(my + t + 1) if t < RS_STEPS else mod(my + (t - RS_STEPS))

        def make_chain(k):
            send_r, recv_r, send_l, recv_l = bufs[4 * k:4 * k + 4]
            ssr, rsr, ssl, rsl = dma_sems[4 * k:4 * k + 4]
            cr, cl = credits[2 * k:2 * k + 2]
            cols = slice(k * NCOL, (k + 1) * NCOL)
            pending = {}

            def prep0():
                send_r[0] = out_ref[top(c_send_r(0)), cols].astype(jnp.bfloat16)
                send_l[0] = out_ref[bot(c_send_l(0)), cols].astype(jnp.bfloat16)

            def start_send(t):
                slot = t % NSLOT
                if t >= NSLOT:
                    pl.semaphore_wait(cr, 1)
                    pl.semaphore_wait(cl, 1)
                rr = pltpu.make_async_remote_copy(
                    src_ref=send_r.at[slot], dst_ref=recv_r.at[slot],
                    send_sem=ssr.at[slot], recv_sem=rsr.at[slot],
                    device_id=(right,), device_id_type=pl.DeviceIdType.MESH,
                )
                rl = pltpu.make_async_remote_copy(
                    src_ref=send_l.at[slot], dst_ref=recv_l.at[slot],
                    send_sem=ssl.at[slot], recv_sem=rsl.at[slot],
                    device_id=(left,), device_id_type=pl.DeviceIdType.MESH,
                )
                rr.start()
                rl.start()
                pending[t] = (rr, rl)

            def step(t):
                rr, rl = pending.pop(t)
                rr.wait()
                rl.wait()
                slot = t % NSLOT
                nslot = (t + 1) % NSLOT
                last = t + 1 >= T_TOTAL
                if t < RS_STEPS:
                    sum_r = (out_ref[top(c_recv_r(t)), cols]
                             + recv_r[slot].astype(jnp.float32))
                    sum_l = (out_ref[bot(c_recv_l(t)), cols]
                             + recv_l[slot].astype(jnp.float32))
                    if t == RS_STEPS - 1:
                        sum_r = _gelu(sum_r)
                        sum_l = _gelu(sum_l)
                    out_ref[top(c_recv_r(t)), cols] = sum_r
                    out_ref[bot(c_recv_l(t)), cols] = sum_l
                    send_r[nslot] = sum_r.astype(jnp.bfloat16)
                    send_l[nslot] = sum_l.astype(jnp.bfloat16)
                else:
                    vr = recv_r[slot]
                    vl = recv_l[slot]
                    out_ref[top(c_recv_r(t)), cols] = vr.astype(jnp.float32)
                    out_ref[bot(c_recv_l(t)), cols] = vl.astype(jnp.float32)
                    if not last:
                        send_r[nslot] = vr
                        send_l[nslot] = vl
                if t + NSLOT < T_TOTAL:
                    pl.semaphore_signal(
                        cr, inc=1,
                        device_id=(left,), device_id_type=pl.DeviceIdType.MESH,
                    )
                    pl.semaphore_signal(
                        cl, inc=1,
                        device_id=(right,), device_id_type=pl.DeviceIdType.MESH,
                    )
                if not last:
                    start_send(t + 1)

            return prep0, start_send, step

        chains = [make_chain(k) for k in range(N_CHAINS)]

        for prep0, _, _ in chains:
            prep0()
        for _, start_send, _ in chains:
            start_send(0)
        for t in range(T_TOTAL):
            for _, _, step in chains:
                step(t)

    comm_buf = pltpu.VMEM((NSLOT, HALF, NCOL), jnp.bfloat16)
    return pl.pallas_call(
        body,
        out_shape=jax.ShapeDtypeStruct((M, N_OUT), jnp.float32),
        in_specs=[
            pl.BlockSpec(memory_space=pltpu.VMEM),
            pl.BlockSpec(memory_space=pltpu.VMEM),
        ],
        out_specs=pl.BlockSpec(memory_space=pltpu.VMEM),
        scratch_shapes=(
            [comm_buf] * (4 * N_CHAINS)
            + [pltpu.SemaphoreType.DMA((NSLOT,))] * (4 * N_CHAINS)
            + [pltpu.SemaphoreType.REGULAR] * (2 * N_CHAINS)
        ),
        compiler_params=pltpu.CompilerParams(
            collective_id=0,
            vmem_limit_bytes=100 * 1024 * 1024,
        ),
    )(x, w_mat)


# device time: 205884 ns/iter; 1.0717x vs baseline; 1.0717x over previous
import jax
import jax.numpy as jnp
from jax import lax
from jax.experimental import pallas as pl
from jax.experimental.pallas import tpu as pltpu

N_DEV = 16
M = 4096
N_OUT = 2048
CHUNK = M // N_DEV
HALF = CHUNK // 2
N_CHAINS = 4
NCOL = N_OUT // N_CHAINS

RS_STEPS = N_DEV - 1
NSLOT = 4
T_TOTAL = 2 * (N_DEV - 1)

_GELU_C = 0.7978845608028654


def _gelu(y):
    return 0.5 * y * (1.0 + jnp.tanh(_GELU_C * (y + 0.044715 * y * y * y)))


def kernel(x, w_mat):
    def body(x_ref, w_ref, out_ref, *scratch):
        bufs = scratch[:4 * N_CHAINS]
        dma_sems = scratch[4 * N_CHAINS:8 * N_CHAINS]
        credits = scratch[8 * N_CHAINS:]

        my = lax.axis_index("i")
        left = lax.rem(my + N_DEV - 1, N_DEV)
        right = lax.rem(my + 1, N_DEV)

        barrier_sem = pltpu.get_barrier_semaphore()
        for nbr in (left, right):
            pl.semaphore_signal(
                barrier_sem, inc=1,
                device_id=(nbr,), device_id_type=pl.DeviceIdType.MESH,
            )
        pl.semaphore_wait(barrier_sem, 2)

        out_ref[...] = jnp.dot(
            x_ref[...].astype(jnp.bfloat16),
            w_ref[...].astype(jnp.bfloat16),
            preferred_element_type=jnp.float32,
        ).astype(jnp.bfloat16)

        def top(c):
            return pl.ds(c * CHUNK, HALF)

        def bot(c):
            return pl.ds(c * CHUNK + HALF, HALF)

        def mod(v):
            return lax.rem(v + 2 * N_DEV, N_DEV)

        def c_send_r(t):
            return mod(my - t) if t < RS_STEPS else mod(my + 1 - (t - RS_STEPS))

        def c_recv_r(t):
            return mod(my - t - 1) if t < RS_STEPS else mod(my - (t - RS_STEPS))

        def c_send_l(t):
            return mod(my + t) if t < RS_STEPS else mod(my - 1 + (t - RS_STEPS))

        def c_recv_l(t):
            return mod(my + t + 1) if t < RS_STEPS else mod(my + (t - RS_STEPS))

        def make_chain(k):
            send_r, recv_r, send_l, recv_l = bufs[4 * k:4 * k + 4]
            ssr, rsr, ssl, rsl = dma_sems[4 * k:4 * k + 4]
            cr, cl = credits[2 * k:2 * k + 2]
            cols = slice(k * NCOL, (k + 1) * NCOL)
            pending = {}

            def prep0():
                send_r[0] = out_ref[top(c_send_r(0)), cols]
                send_l[0] = out_ref[bot(c_send_l(0)), cols]

            def start_send(t):
                slot = t % NSLOT
                if t >= NSLOT:
                    pl.semaphore_wait(cr, 1)
                    pl.semaphore_wait(cl, 1)
                rr = pltpu.make_async_remote_copy(
                    src_ref=send_r.at[slot], dst_ref=recv_r.at[slot],
                    send_sem=ssr.at[slot], recv_sem=rsr.at[slot],
                    device_id=(right,), device_id_type=pl.DeviceIdType.MESH,
                )
                rl = pltpu.make_async_remote_copy(
                    src_ref=send_l.at[slot], dst_ref=recv_l.at[slot],
                    send_sem=ssl.at[slot], recv_sem=rsl.at[slot],
                    device_id=(left,), device_id_type=pl.DeviceIdType.MESH,
                )
                rr.start()
                rl.start()
                pending[t] = (rr, rl)

            def step(t):
                rr, rl = pending.pop(t)
                rr.wait()
                rl.wait()
                slot = t % NSLOT
                nslot = (t + 1) % NSLOT
                last = t + 1 >= T_TOTAL
                if t < RS_STEPS:
                    sum_r = (out_ref[top(c_recv_r(t)), cols].astype(jnp.float32)
                             + recv_r[slot].astype(jnp.float32))
                    sum_l = (out_ref[bot(c_recv_l(t)), cols].astype(jnp.float32)
                             + recv_l[slot].astype(jnp.float32))
                    if t == RS_STEPS - 1:
                        sum_r = _gelu(sum_r)
                        sum_l = _gelu(sum_l)
                    sr = sum_r.astype(jnp.bfloat16)
                    sl = sum_l.astype(jnp.bfloat16)
                    out_ref[top(c_recv_r(t)), cols] = sr
                    out_ref[bot(c_recv_l(t)), cols] = sl
                    send_r[nslot] = sr
                    send_l[nslot] = sl
                else:
                    vr = recv_r[slot]
                    vl = recv_l[slot]
                    out_ref[top(c_recv_r(t)), cols] = vr
                    out_ref[bot(c_recv_l(t)), cols] = vl
                    if not last:
                        send_r[nslot] = vr
                        send_l[nslot] = vl
                if t + NSLOT < T_TOTAL:
                    pl.semaphore_signal(
                        cr, inc=1,
                        device_id=(left,), device_id_type=pl.DeviceIdType.MESH,
                    )
                    pl.semaphore_signal(
                        cl, inc=1,
                        device_id=(right,), device_id_type=pl.DeviceIdType.MESH,
                    )
                if not last:
                    start_send(t + 1)

            return prep0, start_send, step

        chains = [make_chain(k) for k in range(N_CHAINS)]

        for prep0, _, _ in chains:
            prep0()
        for _, start_send, _ in chains:
            start_send(0)
        for t in range(T_TOTAL):
            for _, _, step in chains:
                step(t)

    comm_buf = pltpu.VMEM((NSLOT, HALF, NCOL), jnp.bfloat16)
    return pl.pallas_call(
        body,
        out_shape=jax.ShapeDtypeStruct((M, N_OUT), jnp.bfloat16),
        in_specs=[
            pl.BlockSpec(memory_space=pltpu.VMEM),
            pl.BlockSpec(memory_space=pltpu.VMEM),
        ],
        out_specs=pl.BlockSpec(memory_space=pltpu.VMEM),
        scratch_shapes=(
            [comm_buf] * (4 * N_CHAINS)
            + [pltpu.SemaphoreType.DMA((NSLOT,))] * (4 * N_CHAINS)
            + [pltpu.SemaphoreType.REGULAR] * (2 * N_CHAINS)
        ),
        compiler_params=pltpu.CompilerParams(
            collective_id=0,
            vmem_limit_bytes=100 * 1024 * 1024,
        ),
    )(x, w_mat)


# device time: 201346 ns/iter; 1.0958x vs baseline; 1.0225x over previous
import jax
import jax.numpy as jnp
from jax import lax
from jax.experimental import pallas as pl
from jax.experimental.pallas import tpu as pltpu

N_DEV = 16
M = 4096
N_OUT = 2048
CHUNK = M // N_DEV
HALF = CHUNK // 2
N_CHAINS = 4
NCOL = N_OUT // N_CHAINS

RS_STEPS = N_DEV - 1
NSLOT = 4
T_TOTAL = 2 * (N_DEV - 1)

_GELU_C = 0.7978845608028654


def _gelu(y):
    return 0.5 * y * (1.0 + jnp.tanh(_GELU_C * (y + 0.044715 * y * y * y)))


def kernel(x, w_mat):
    def body(x_ref, w_ref, out_ref, *scratch):
        bufs = scratch[:4 * N_CHAINS]
        dma_sems = scratch[4 * N_CHAINS:8 * N_CHAINS]
        credits = scratch[8 * N_CHAINS:]

        my = lax.axis_index("i")
        left = lax.rem(my + N_DEV - 1, N_DEV)
        right = lax.rem(my + 1, N_DEV)

        barrier_sem = pltpu.get_barrier_semaphore()
        for nbr in (left, right):
            pl.semaphore_signal(
                barrier_sem, inc=1,
                device_id=(nbr,), device_id_type=pl.DeviceIdType.MESH,
            )
        pl.semaphore_wait(barrier_sem, 2)

        w_bf = w_ref[...].astype(jnp.bfloat16)
        rows0 = pl.ds(my * CHUNK, CHUNK)
        out_ref[rows0, :] = jnp.dot(
            x_ref[rows0, :].astype(jnp.bfloat16),
            w_bf,
            preferred_element_type=jnp.float32,
        ).astype(jnp.bfloat16)

        def top(c):
            return pl.ds(c * CHUNK, HALF)

        def bot(c):
            return pl.ds(c * CHUNK + HALF, HALF)

        def mod(v):
            return lax.rem(v + 2 * N_DEV, N_DEV)

        def c_send_r(t):
            return mod(my - t) if t < RS_STEPS else mod(my + 1 - (t - RS_STEPS))

        def c_recv_r(t):
            return mod(my - t - 1) if t < RS_STEPS else mod(my - (t - RS_STEPS))

        def c_send_l(t):
            return mod(my + t) if t < RS_STEPS else mod(my - 1 + (t - RS_STEPS))

        def c_recv_l(t):
            return mod(my + t + 1) if t < RS_STEPS else mod(my + (t - RS_STEPS))

        def make_chain(k):
            send_r, recv_r, send_l, recv_l = bufs[4 * k:4 * k + 4]
            ssr, rsr, ssl, rsl = dma_sems[4 * k:4 * k + 4]
            cr, cl = credits[2 * k:2 * k + 2]
            cols = slice(k * NCOL, (k + 1) * NCOL)
            pending = {}

            def prep0():
                send_r[0] = out_ref[top(c_send_r(0)), cols]
                send_l[0] = out_ref[bot(c_send_l(0)), cols]

            def start_send(t):
                slot = t % NSLOT
                if t >= NSLOT:
                    pl.semaphore_wait(cr, 1)
                    pl.semaphore_wait(cl, 1)
                rr = pltpu.make_async_remote_copy(
                    src_ref=send_r.at[slot], dst_ref=recv_r.at[slot],
                    send_sem=ssr.at[slot], recv_sem=rsr.at[slot],
                    device_id=(right,), device_id_type=pl.DeviceIdType.MESH,
                )
                rl = pltpu.make_async_remote_copy(
                    src_ref=send_l.at[slot], dst_ref=recv_l.at[slot],
                    send_sem=ssl.at[slot], recv_sem=rsl.at[slot],
                    device_id=(left,), device_id_type=pl.DeviceIdType.MESH,
                )
                rr.start()
                rl.start()
                pending[t] = (rr, rl)

            def step(t):
                rr, rl = pending.pop(t)
                rr.wait()
                rl.wait()
                slot = t % NSLOT
                nslot = (t + 1) % NSLOT
                last = t + 1 >= T_TOTAL
                if t < RS_STEPS:
                    sum_r = (out_ref[top(c_recv_r(t)), cols].astype(jnp.float32)
                             + recv_r[slot].astype(jnp.float32))
                    sum_l = (out_ref[bot(c_recv_l(t)), cols].astype(jnp.float32)
                             + recv_l[slot].astype(jnp.float32))
                    if t == RS_STEPS - 1:
                        sum_r = _gelu(sum_r)
                        sum_l = _gelu(sum_l)
                    sr = sum_r.astype(jnp.bfloat16)
                    sl = sum_l.astype(jnp.bfloat16)
                    out_ref[top(c_recv_r(t)), cols] = sr
                    out_ref[bot(c_recv_l(t)), cols] = sl
                    send_r[nslot] = sr
                    send_l[nslot] = sl
                else:
                    vr = recv_r[slot]
                    vl = recv_l[slot]
                    out_ref[top(c_recv_r(t)), cols] = vr
                    out_ref[bot(c_recv_l(t)), cols] = vl
                    if not last:
                        send_r[nslot] = vr
                        send_l[nslot] = vl
                if t + NSLOT < T_TOTAL:
                    pl.semaphore_signal(
                        cr, inc=1,
                        device_id=(left,), device_id_type=pl.DeviceIdType.MESH,
                    )
                    pl.semaphore_signal(
                        cl, inc=1,
                        device_id=(right,), device_id_type=pl.DeviceIdType.MESH,
                    )
                if not last:
                    start_send(t + 1)

            return prep0, start_send, step

        chains = [make_chain(k) for k in range(N_CHAINS)]

        for prep0, _, _ in chains:
            prep0()
        for _, start_send, _ in chains:
            start_send(0)
        out_ref[...] = jnp.dot(
            x_ref[...].astype(jnp.bfloat16),
            w_bf,
            preferred_element_type=jnp.float32,
        ).astype(jnp.bfloat16)
        for t in range(T_TOTAL):
            for _, _, step in chains:
                step(t)

    comm_buf = pltpu.VMEM((NSLOT, HALF, NCOL), jnp.bfloat16)
    return pl.pallas_call(
        body,
        out_shape=jax.ShapeDtypeStruct((M, N_OUT), jnp.bfloat16),
        in_specs=[
            pl.BlockSpec(memory_space=pltpu.VMEM),
            pl.BlockSpec(memory_space=pltpu.VMEM),
        ],
        out_specs=pl.BlockSpec(memory_space=pltpu.VMEM),
        scratch_shapes=(
            [comm_buf] * (4 * N_CHAINS)
            + [pltpu.SemaphoreType.DMA((NSLOT,))] * (4 * N_CHAINS)
            + [pltpu.SemaphoreType.REGULAR] * (2 * N_CHAINS)
        ),
        compiler_params=pltpu.CompilerParams(
            collective_id=0,
            vmem_limit_bytes=100 * 1024 * 1024,
        ),
    )(x, w_mat)
